# baseline (speedup 1.0000x reference)
import functools

import jax
import jax.numpy as jnp
from jax import lax
from jax.experimental import pallas as pl
from jax.experimental.pallas import tpu as pltpu
from jax.experimental.pallas import tpu_sc as plsc

F = 26
D = 32
B = 16384
E = 130
_OFFSETS = (1, 5, 7, 11, 13)

_PAIRS = sorted(((i + o) % F, i) for i in range(F) for o in _OFFSETS)
_K = {p: k for k, p in enumerate(_PAIRS)}
_SRC = tuple(tuple((f + o) % F for o in _OFFSETS) for f in range(F))
_WIDX = tuple(tuple(_K[(s, f)] for s in _SRC[f]) for f in range(F))

_NW = 32
_ROWS_PER_W = B // _NW
_NB = 16
_NCHUNK = _ROWS_PER_W // _NB


_EPAD = 144


def _tanh(x):
    return 1.0 - 2.0 / (jnp.exp(x + x) + 1.0)


def _layer(srcs, ws):
    outs = []
    for f in range(F):
        acc = None
        for t in range(len(_OFFSETS)):
            term = srcs[_SRC[f][t]] * ws[_WIDX[f][t]]
            acc = term if acc is None else acc + term
        outs.append(_tanh(acc))
    return outs


_ROW = F * D


@functools.partial(
    pl.kernel,
    mesh=plsc.VectorSubcoreMesh(core_axis_name="c", subcore_axis_name="s"),
    out_type=jax.ShapeDtypeStruct((B, _ROW), jnp.float32),
    scratch_types=[
        pltpu.VMEM((_EPAD,), jnp.float32),
        pltpu.VMEM((_EPAD,), jnp.float32),
        pltpu.VMEM((_NB, _ROW), jnp.float32),
        pltpu.VMEM((_NB, _ROW), jnp.float32),
        pltpu.VMEM((_NB, _ROW), jnp.float32),
        pltpu.VMEM((_NB, _ROW), jnp.float32),
        pltpu.SemaphoreType.DMA,
        pltpu.SemaphoreType.DMA,
        pltpu.SemaphoreType.DMA,
        pltpu.SemaphoreType.DMA,
    ],
)
def _gnn_sc(in_hbm, w0_hbm, w1_hbm, out_hbm, w0_v, w1_v,
            in_v0, in_v1, out_v0, out_v1, si0, si1, so0, so1):
    wid = lax.axis_index("s") * 2 + lax.axis_index("c")
    base = wid * _ROWS_PER_W
    in_bufs = (in_v0, in_v1)
    out_bufs = (out_v0, out_v1)
    si = (si0, si1)
    so = (so0, so1)

    pltpu.sync_copy(w0_hbm, w0_v.at[pl.ds(0, E)])
    pltpu.sync_copy(w1_hbm, w1_v.at[pl.ds(0, E)])
    wch = [[wv[pl.ds(16 * c, 16)] for c in range(_EPAD // 16)]
           for wv in (w0_v, w1_v)]
    ws = [[wch[l][k // 16][k % 16] for k in range(E)] for l in (0, 1)]

    def compute_chunk(in_v, out_v):
        def row_body(b, carry):
            for j in range(D // 16):
                srcs = [in_v[b, pl.ds(f * D + 16 * j, 16)]
                        for f in range(F)]
                mids = _layer(srcs, ws[0])
                finals = _layer(mids, ws[1])
                for f in range(F):
                    out_v[b, pl.ds(f * D + 16 * j, 16)] = finals[f]
            return carry

        lax.fori_loop(0, _NB, row_body, 0)

    def in_slice(ci):
        return in_hbm.at[pl.ds(base + ci * _NB, _NB)]

    def out_slice(ci):
        return out_hbm.at[pl.ds(base + ci * _NB, _NB)]

    pltpu.async_copy(in_slice(0), in_bufs[0], si[0])
    pltpu.async_copy(in_slice(1), in_bufs[1], si[1])

    @pl.loop(0, _NCHUNK, step=2)
    def chunk_loop(g):
        for b in range(2):
            ci = g + b
            pltpu.make_async_copy(in_slice(ci), in_bufs[b], si[b]).wait()

            @pl.when(ci >= 2)
            def _():
                pltpu.make_async_copy(
                    out_bufs[b], out_slice(ci), so[b]).wait()

            compute_chunk(in_bufs[b], out_bufs[b])
            pltpu.async_copy(out_bufs[b], out_slice(ci), so[b])

            @pl.when(ci + 2 < _NCHUNK)
            def _():
                pltpu.async_copy(in_slice(ci + 2), in_bufs[b], si[b])

    for b in range(2):
        pltpu.make_async_copy(
            out_bufs[b], out_slice(_NCHUNK - 2 + b), so[b]).wait()


def kernel(inputs, w0, w1):
    return _gnn_sc(inputs.reshape(B, _ROW), w0, w1)


# speedup vs baseline: 1.7358x; 1.7358x over previous
import functools

import jax
import jax.numpy as jnp
from jax import lax
from jax.experimental import pallas as pl
from jax.experimental.pallas import tpu as pltpu
from jax.experimental.pallas import tpu_sc as plsc

F = 26
D = 32
B = 16384
E = 130
_OFFSETS = (1, 5, 7, 11, 13)

_PAIRS = sorted(((i + o) % F, i) for i in range(F) for o in _OFFSETS)
_K = {p: k for k, p in enumerate(_PAIRS)}
_SRC = tuple(tuple((f + o) % F for o in _OFFSETS) for f in range(F))
_WIDX = tuple(tuple(_K[(s, f)] for s in _SRC[f]) for f in range(F))

_NW = 32
_ROWS_PER_W = B // _NW
_NB = 16
_NCHUNK = _ROWS_PER_W // _NB


_EPAD = 144


def _tanh(x):
    return 1.0 - 2.0 / (jnp.exp(2.0 * x) + 1.0)


def _layer(srcs, ws):
    outs = []
    for f in range(F):
        t0, t1, t2, t3, t4 = (
            srcs[_SRC[f][t]] * ws[_WIDX[f][t]] for t in range(5))
        outs.append(_tanh(((t0 + t1) + (t2 + t3)) + t4))
    return outs


_ROW = F * D


@functools.partial(
    pl.kernel,
    mesh=plsc.VectorSubcoreMesh(core_axis_name="c", subcore_axis_name="s"),
    out_type=jax.ShapeDtypeStruct((B, _ROW), jnp.float32),
    scratch_types=[
        pltpu.VMEM((_EPAD,), jnp.float32),
        pltpu.VMEM((_EPAD,), jnp.float32),
        pltpu.VMEM((_NB, _ROW), jnp.float32),
        pltpu.VMEM((_NB, _ROW), jnp.float32),
        pltpu.VMEM((_NB, _ROW), jnp.float32),
        pltpu.VMEM((_NB, _ROW), jnp.float32),
        pltpu.SemaphoreType.DMA,
        pltpu.SemaphoreType.DMA,
        pltpu.SemaphoreType.DMA,
        pltpu.SemaphoreType.DMA,
    ],
)
def _gnn_sc(in_hbm, w0_hbm, w1_hbm, out_hbm, w0_v, w1_v,
            in_v0, in_v1, out_v0, out_v1, si0, si1, so0, so1):
    wid = lax.axis_index("s") * 2 + lax.axis_index("c")
    base = wid * _ROWS_PER_W
    in_bufs = (in_v0, in_v1)
    out_bufs = (out_v0, out_v1)
    si = (si0, si1)
    so = (so0, so1)

    pltpu.sync_copy(w0_hbm, w0_v.at[pl.ds(0, E)])
    pltpu.sync_copy(w1_hbm, w1_v.at[pl.ds(0, E)])
    wch = [[wv[pl.ds(16 * c, 16)] for c in range(_EPAD // 16)]
           for wv in (w0_v, w1_v)]
    ws = [[wch[l][k // 16][k % 16] for k in range(E)] for l in (0, 1)]

    def compute_chunk(in_v, out_v):
        def row_body(b, carry):
            for j in range(D // 16):
                srcs = [in_v[b, pl.ds(f * D + 16 * j, 16)]
                        for f in range(F)]
                mids = _layer(srcs, ws[0])
                finals = _layer(mids, ws[1])
                for f in range(F):
                    out_v[b, pl.ds(f * D + 16 * j, 16)] = finals[f]
            return carry

        lax.fori_loop(0, _NB, row_body, 0)

    def in_slice(ci):
        return in_hbm.at[pl.ds(base + ci * _NB, _NB)]

    def out_slice(ci):
        return out_hbm.at[pl.ds(base + ci * _NB, _NB)]

    pltpu.async_copy(in_slice(0), in_bufs[0], si[0])
    pltpu.async_copy(in_slice(1), in_bufs[1], si[1])

    @pl.loop(0, _NCHUNK, step=2)
    def chunk_loop(g):
        for b in range(2):
            ci = g + b
            pltpu.make_async_copy(in_slice(ci), in_bufs[b], si[b]).wait()

            @pl.when(ci >= 2)
            def _():
                pltpu.make_async_copy(
                    out_bufs[b], out_slice(ci), so[b]).wait()

            compute_chunk(in_bufs[b], out_bufs[b])
            pltpu.async_copy(out_bufs[b], out_slice(ci), so[b])

            @pl.when(ci + 2 < _NCHUNK)
            def _():
                pltpu.async_copy(in_slice(ci + 2), in_bufs[b], si[b])

    for b in range(2):
        pltpu.make_async_copy(
            out_bufs[b], out_slice(_NCHUNK - 2 + b), so[b]).wait()


def kernel(inputs, w0, w1):
    return _gnn_sc(inputs.reshape(B, _ROW), w0, w1)
